# baseline (speedup 1.0000x reference)
import functools

import jax
import jax.numpy as jnp
from jax import lax
from jax.experimental import pallas as pl
from jax.experimental.pallas import tpu as pltpu
from jax.experimental.pallas import tpu_sc as plsc

_NC = 2
_NS = 16
_NW = _NC * _NS
_CW = 128


def _sc_gather(uidx, iidx, user_table, item_table, hidden):
    batch = uidx.shape[0] * uidx.shape[1] * uidx.shape[2]
    b_per_w = batch // _NW
    n_chunks = b_per_w // _CW
    mesh = plsc.VectorSubcoreMesh(core_axis_name="c", subcore_axis_name="s")

    @functools.partial(
        pl.kernel,
        mesh=mesh,
        compiler_params=pltpu.CompilerParams(use_tc_tiling_on_sc=False),
        out_type=(
            jax.ShapeDtypeStruct((batch, hidden), jnp.float32),
            jax.ShapeDtypeStruct((batch, hidden), jnp.float32),
        ),
        scratch_types=(
            pltpu.VMEM((n_chunks, _CW), jnp.int32),
            pltpu.VMEM((n_chunks, _CW), jnp.int32),
            pltpu.VMEM((b_per_w, hidden), jnp.float32),
            pltpu.VMEM((b_per_w, hidden), jnp.float32),
            pltpu.SemaphoreType.DMA,
        ),
    )
    def gather_kernel(uidx_hbm, iidx_hbm, utab_hbm, itab_hbm,
                      uout_hbm, iout_hbm,
                      idx_u, idx_i, rows_u, rows_i, sem):
        wid = lax.axis_index("s") * _NC + lax.axis_index("c")
        pltpu.sync_copy(uidx_hbm.at[wid], idx_u)
        pltpu.sync_copy(iidx_hbm.at[wid], idx_i)
        copies = []
        for j in range(n_chunks):
            copies.append(pltpu.async_copy(
                utab_hbm.at[idx_u.at[j]],
                rows_u.at[pl.ds(j * _CW, _CW)], sem))
            copies.append(pltpu.async_copy(
                itab_hbm.at[idx_i.at[j]],
                rows_i.at[pl.ds(j * _CW, _CW)], sem))
        for c in copies:
            c.wait()
        base = wid * b_per_w
        pltpu.sync_copy(rows_u, uout_hbm.at[pl.ds(base, b_per_w)])
        pltpu.sync_copy(rows_i, iout_hbm.at[pl.ds(base, b_per_w)])

    return gather_kernel(uidx, iidx, user_table, item_table)


def _mlp_body(u, i, w1u, w1i, b1, w2, b2, w3, b3, o):
    h = jnp.dot(u[...], w1u[...], preferred_element_type=jnp.float32)
    h = h + jnp.dot(i[...], w1i[...], preferred_element_type=jnp.float32)
    h = jnp.maximum(h + b1[...], 0.0)
    h = jnp.maximum(
        jnp.dot(h, w2[...], preferred_element_type=jnp.float32) + b2[...], 0.0)
    o[...] = jnp.dot(h, w3[...], preferred_element_type=jnp.float32) + b3[...]


def _tc_mlp(uemb, iemb, w1u, w1i, b1, w2, b2, w3, b3, blk):
    batch, hidden = uemb.shape
    d2 = w2.shape[1]
    d3 = w3.shape[1]
    grid = (batch // blk,)
    fixed = lambda b: (0, 0)
    return pl.pallas_call(
        _mlp_body,
        grid=grid,
        in_specs=[
            pl.BlockSpec((blk, hidden), lambda b: (b, 0)),
            pl.BlockSpec((blk, hidden), lambda b: (b, 0)),
            pl.BlockSpec((hidden, hidden), fixed),
            pl.BlockSpec((hidden, hidden), fixed),
            pl.BlockSpec((1, hidden), fixed),
            pl.BlockSpec((hidden, d2), fixed),
            pl.BlockSpec((1, d2), fixed),
            pl.BlockSpec((d2, d3), fixed),
            pl.BlockSpec((1, d3), fixed),
        ],
        out_specs=pl.BlockSpec((blk, d3), lambda b: (b, 0)),
        out_shape=jax.ShapeDtypeStruct((batch, d3), jnp.float32),
    )(uemb, iemb, w1u, w1i, b1, w2, b2, w3, b3)


def kernel(user_id, item_id, user_table, item_table, W1, b1, W2, b2, W3, b3):
    batch = user_id.shape[0]
    hidden = user_table.shape[1]
    b_per_w = batch // _NW
    n_chunks = b_per_w // _CW
    uidx = user_id.astype(jnp.int32).reshape(_NW, n_chunks, _CW)
    iidx = item_id.astype(jnp.int32).reshape(_NW, n_chunks, _CW)
    uemb, iemb = _sc_gather(uidx, iidx, user_table, item_table, hidden)
    return _tc_mlp(
        uemb, iemb,
        W1[:hidden], W1[hidden:], b1.reshape(1, -1),
        W2, b2.reshape(1, -1), W3, b3.reshape(1, -1),
        blk=2048,
    )


# speedup vs baseline: 2.4248x; 2.4248x over previous
import functools

import jax
import jax.numpy as jnp
from jax import lax
from jax.experimental import pallas as pl
from jax.experimental.pallas import tpu as pltpu
from jax.experimental.pallas import tpu_sc as plsc

_NC = 2
_NS = 16
_NW = _NC * _NS
_GRP = 16
_DEPTH = 4
_FLUSH = 64


def _sc_gather_concat(uidx, iidx, utab3, itab3):
    batch = uidx.shape[0]
    n_tr, n_sub, _ = utab3.shape
    hidden = n_tr * n_sub
    b_per_w = batch // _NW
    n_grp = b_per_w // _GRP
    mesh = plsc.VectorSubcoreMesh(core_axis_name="c", subcore_axis_name="s")

    block = pltpu.VMEM((n_tr, n_sub, 128), jnp.float32)

    @functools.partial(
        pl.kernel,
        mesh=mesh,
        compiler_params=pltpu.CompilerParams(needs_layout_passes=False),
        out_type=jax.ShapeDtypeStruct((batch, 2 * hidden), jnp.float32),
        scratch_types=(
            pltpu.VMEM((b_per_w,), jnp.int32),
            pltpu.VMEM((b_per_w,), jnp.int32),
            [block] * _DEPTH,
            [block] * _DEPTH,
            pltpu.VMEM((_FLUSH, 2 * hidden), jnp.float32),
            [pltpu.SemaphoreType.DMA] * _DEPTH,
        ),
    )
    def gather_kernel(uidx_hbm, iidx_hbm, utab_hbm, itab_hbm, out_hbm,
                      idx_u, idx_i, bus, bis, stage, sems):
        wid = lax.axis_index("s") * _NC + lax.axis_index("c")
        base = wid * b_per_w
        pltpu.sync_copy(uidx_hbm.at[pl.ds(base, b_per_w)], idx_u)
        pltpu.sync_copy(iidx_hbm.at[pl.ds(base, b_per_w)], idx_i)

        lanes = lax.iota(jnp.int32, 16)
        trq = [(lanes + 16 * q) >> 3 for q in range(hidden // 16)]
        sq = [(lanes + 16 * q) & 7 for q in range(hidden // 16)]

        def fire(uid, iid, d):
            uoff = pl.multiple_of((uid >> 7) * 128, 128)
            ioff = pl.multiple_of((iid >> 7) * 128, 128)
            pltpu.async_copy(utab_hbm.at[:, :, pl.ds(uoff, 128)], bus[d],
                             sems[d])
            pltpu.async_copy(itab_hbm.at[:, :, pl.ds(ioff, 128)], bis[d],
                             sems[d])

        def wait_pair(d):
            pltpu.make_async_copy(utab_hbm.at[:, :, pl.ds(0, 128)], bus[d],
                                  sems[d]).wait()
            pltpu.make_async_copy(itab_hbm.at[:, :, pl.ds(0, 128)], bis[d],
                                  sems[d]).wait()

        def extract(uid, iid, r, d):
            lu = lanes * 0 + (uid & 127)
            li = lanes * 0 + (iid & 127)
            for q in range(hidden // 16):
                vu = plsc.load_gather(bus[d], [trq[q], sq[q], lu])
                vi = plsc.load_gather(bis[d], [trq[q], sq[q], li])
                stage[r, pl.ds(16 * q, 16)] = vu
                stage[r, pl.ds(hidden + 16 * q, 16)] = vi

        def group_body(g, carry):
            vu = idx_u[pl.ds(g * _GRP, _GRP)]
            vi = idx_i[pl.ds(g * _GRP, _GRP)]
            uids = [vu[l] for l in range(_GRP)]
            iids = [vi[l] for l in range(_GRP)]
            for l in range(_DEPTH):
                fire(uids[l], iids[l], l)
            for l in range(_GRP):
                d = l % _DEPTH
                wait_pair(d)
                extract(uids[l], iids[l], (g * _GRP + l) & (_FLUSH - 1), d)
                if l + _DEPTH < _GRP:
                    fire(uids[l + _DEPTH], iids[l + _DEPTH], d)

            @pl.when((g & 3) == 3)
            def _():
                row0 = pl.multiple_of(base + ((g >> 2) << 6), _FLUSH)
                pltpu.sync_copy(stage, out_hbm.at[pl.ds(row0, _FLUSH), :])

            return carry

        lax.fori_loop(0, n_grp, group_body, 0)

    return gather_kernel(uidx, iidx, utab3, itab3)


def _mlp_body(x_ref, w1_ref, b1_ref, w2_ref, b2_ref, w3_ref, b3_ref, o_ref):
    h = jnp.dot(x_ref[...], w1_ref[...], preferred_element_type=jnp.float32)
    h = jnp.maximum(h + b1_ref[...], 0.0)
    h = jnp.maximum(
        jnp.dot(h, w2_ref[...], preferred_element_type=jnp.float32)
        + b2_ref[...], 0.0)
    o_ref[...] = (jnp.dot(h, w3_ref[...], preferred_element_type=jnp.float32)
                  + b3_ref[...])


def _tc_mlp(x, w1, b1r, w2, b2r, w3, b3r, blk):
    batch, d_in = x.shape
    d1 = w1.shape[1]
    d2 = w2.shape[1]
    d3 = w3.shape[1]
    grid = (batch // blk,)
    fixed = lambda b: (0, 0)
    return pl.pallas_call(
        _mlp_body,
        grid=grid,
        in_specs=[
            pl.BlockSpec((blk, d_in), lambda b: (b, 0)),
            pl.BlockSpec((d_in, d1), fixed),
            pl.BlockSpec((1, d1), fixed),
            pl.BlockSpec((d1, d2), fixed),
            pl.BlockSpec((1, d2), fixed),
            pl.BlockSpec((d2, d3), fixed),
            pl.BlockSpec((1, d3), fixed),
        ],
        out_specs=pl.BlockSpec((blk, d3), lambda b: (b, 0)),
        out_shape=jax.ShapeDtypeStruct((batch, d3), jnp.float32),
    )(x, w1, b1r, w2, b2r, w3, b3r)


def kernel(user_id, item_id, user_table, item_table, W1, b1, W2, b2, W3, b3):
    n_ids, hidden = user_table.shape
    utab3 = user_table.T.reshape(8, hidden // 8, n_ids)
    itab3 = item_table.T.reshape(8, hidden // 8, n_ids)
    x = _sc_gather_concat(
        user_id.astype(jnp.int32), item_id.astype(jnp.int32), utab3, itab3)
    return _tc_mlp(
        x, W1, b1.reshape(1, -1), W2, b2.reshape(1, -1),
        W3, b3.reshape(1, -1), blk=2048)


# speedup vs baseline: 2.4885x; 1.0263x over previous
import functools

import jax
import jax.numpy as jnp
from jax import lax
from jax.experimental import pallas as pl
from jax.experimental.pallas import tpu as pltpu
from jax.experimental.pallas import tpu_sc as plsc

_NC = 2
_NS = 16
_NW = _NC * _NS
_GRP = 16
_DEPTH = 6
_FLUSH = 64


def _sc_gather_concat(uidx, iidx, utab3, itab3):
    batch = uidx.shape[0]
    n_tr, n_sub, _ = utab3.shape
    hidden = n_tr * n_sub
    b_per_w = batch // _NW
    n_grp = b_per_w // _GRP
    mesh = plsc.VectorSubcoreMesh(core_axis_name="c", subcore_axis_name="s")

    block = pltpu.VMEM((n_tr, n_sub, 128), jnp.float32)

    @functools.partial(
        pl.kernel,
        mesh=mesh,
        compiler_params=pltpu.CompilerParams(needs_layout_passes=False),
        out_type=jax.ShapeDtypeStruct((batch, 2 * hidden), jnp.float32),
        scratch_types=(
            pltpu.VMEM((b_per_w,), jnp.int32),
            pltpu.VMEM((b_per_w,), jnp.int32),
            [block] * _DEPTH,
            [block] * _DEPTH,
            pltpu.VMEM((_FLUSH, 2 * hidden), jnp.float32),
            [pltpu.SemaphoreType.DMA] * _DEPTH,
        ),
    )
    def gather_kernel(uidx_hbm, iidx_hbm, utab_hbm, itab_hbm, out_hbm,
                      idx_u, idx_i, bus, bis, stage, sems):
        wid = lax.axis_index("s") * _NC + lax.axis_index("c")
        base = wid * b_per_w
        pltpu.sync_copy(uidx_hbm.at[pl.ds(base, b_per_w)], idx_u)
        pltpu.sync_copy(iidx_hbm.at[pl.ds(base, b_per_w)], idx_i)

        lanes = lax.iota(jnp.int32, 16)
        trq = [(lanes + 16 * q) >> 3 for q in range(hidden // 16)]
        sq = [(lanes + 16 * q) & 7 for q in range(hidden // 16)]

        def fire(uid, iid, d):
            uoff = pl.multiple_of((uid >> 7) * 128, 128)
            ioff = pl.multiple_of((iid >> 7) * 128, 128)
            pltpu.async_copy(utab_hbm.at[:, :, pl.ds(uoff, 128)], bus[d],
                             sems[d])
            pltpu.async_copy(itab_hbm.at[:, :, pl.ds(ioff, 128)], bis[d],
                             sems[d])

        def wait_pair(d):
            pltpu.make_async_copy(utab_hbm.at[:, :, pl.ds(0, 128)], bus[d],
                                  sems[d]).wait()
            pltpu.make_async_copy(itab_hbm.at[:, :, pl.ds(0, 128)], bis[d],
                                  sems[d]).wait()

        def extract(uid, iid, r, d):
            lu = lanes * 0 + (uid & 127)
            li = lanes * 0 + (iid & 127)
            for q in range(hidden // 16):
                vu = plsc.load_gather(bus[d], [trq[q], sq[q], lu])
                vi = plsc.load_gather(bis[d], [trq[q], sq[q], li])
                stage[r, pl.ds(16 * q, 16)] = vu
                stage[r, pl.ds(hidden + 16 * q, 16)] = vi

        def group_body(g, carry):
            vu = idx_u[pl.ds(g * _GRP, _GRP)]
            vi = idx_i[pl.ds(g * _GRP, _GRP)]
            uids = [vu[l] for l in range(_GRP)]
            iids = [vi[l] for l in range(_GRP)]
            for l in range(_DEPTH):
                fire(uids[l], iids[l], l)
            for l in range(_GRP):
                d = l % _DEPTH
                wait_pair(d)
                extract(uids[l], iids[l], (g * _GRP + l) & (_FLUSH - 1), d)
                if l + _DEPTH < _GRP:
                    fire(uids[l + _DEPTH], iids[l + _DEPTH], d)

            @pl.when((g & 3) == 3)
            def _():
                row0 = pl.multiple_of(base + ((g >> 2) << 6), _FLUSH)
                pltpu.sync_copy(stage, out_hbm.at[pl.ds(row0, _FLUSH), :])

            return carry

        lax.fori_loop(0, n_grp, group_body, 0)

    return gather_kernel(uidx, iidx, utab3, itab3)


def _mlp_body(x_ref, w1_ref, b1_ref, w2_ref, b2_ref, w3_ref, b3_ref, o_ref):
    h = jnp.dot(x_ref[...], w1_ref[...], preferred_element_type=jnp.float32)
    h = jnp.maximum(h + b1_ref[...], 0.0)
    h = jnp.maximum(
        jnp.dot(h, w2_ref[...], preferred_element_type=jnp.float32)
        + b2_ref[...], 0.0)
    o_ref[...] = (jnp.dot(h, w3_ref[...], preferred_element_type=jnp.float32)
                  + b3_ref[...])


def _tc_mlp(x, w1, b1r, w2, b2r, w3, b3r, blk):
    batch, d_in = x.shape
    d1 = w1.shape[1]
    d2 = w2.shape[1]
    d3 = w3.shape[1]
    grid = (batch // blk,)
    fixed = lambda b: (0, 0)
    return pl.pallas_call(
        _mlp_body,
        grid=grid,
        in_specs=[
            pl.BlockSpec((blk, d_in), lambda b: (b, 0)),
            pl.BlockSpec((d_in, d1), fixed),
            pl.BlockSpec((1, d1), fixed),
            pl.BlockSpec((d1, d2), fixed),
            pl.BlockSpec((1, d2), fixed),
            pl.BlockSpec((d2, d3), fixed),
            pl.BlockSpec((1, d3), fixed),
        ],
        out_specs=pl.BlockSpec((blk, d3), lambda b: (b, 0)),
        out_shape=jax.ShapeDtypeStruct((batch, d3), jnp.float32),
    )(x, w1, b1r, w2, b2r, w3, b3r)


def kernel(user_id, item_id, user_table, item_table, W1, b1, W2, b2, W3, b3):
    n_ids, hidden = user_table.shape
    utab3 = user_table.T.reshape(8, hidden // 8, n_ids)
    itab3 = item_table.T.reshape(8, hidden // 8, n_ids)
    x = _sc_gather_concat(
        user_id.astype(jnp.int32), item_id.astype(jnp.int32), utab3, itab3)
    return _tc_mlp(
        x, W1, b1.reshape(1, -1), W2, b2.reshape(1, -1),
        W3, b3.reshape(1, -1), blk=2048)


# speedup vs baseline: 2.6090x; 1.0484x over previous
import functools

import jax
import jax.numpy as jnp
from jax import lax
from jax.experimental import pallas as pl
from jax.experimental.pallas import tpu as pltpu
from jax.experimental.pallas import tpu_sc as plsc

_NC = 2
_NS = 16
_NW = _NC * _NS
_GRP = 16
_DEPTH = 7
_FLUSH = 64


def _sc_gather_concat(uidx, iidx, utab3, itab3):
    batch = uidx.shape[0]
    n_tr, n_sub, _ = utab3.shape
    hidden = n_tr * n_sub
    b_per_w = batch // _NW
    n_grp = b_per_w // _GRP
    mesh = plsc.VectorSubcoreMesh(core_axis_name="c", subcore_axis_name="s")

    block = pltpu.VMEM((n_tr, n_sub, 128), jnp.float32)

    @functools.partial(
        pl.kernel,
        mesh=mesh,
        compiler_params=pltpu.CompilerParams(needs_layout_passes=False),
        out_type=jax.ShapeDtypeStruct((batch, 2 * hidden), jnp.float32),
        scratch_types=(
            pltpu.VMEM((b_per_w,), jnp.int32),
            pltpu.VMEM((b_per_w,), jnp.int32),
            [block] * _DEPTH,
            [block] * _DEPTH,
            pltpu.VMEM((_FLUSH, 2 * hidden), jnp.float32),
            [pltpu.SemaphoreType.DMA] * _DEPTH,
        ),
    )
    def gather_kernel(uidx_hbm, iidx_hbm, utab_hbm, itab_hbm, out_hbm,
                      idx_u, idx_i, bus, bis, stage, sems):
        wid = lax.axis_index("s") * _NC + lax.axis_index("c")
        base = wid * b_per_w
        pltpu.sync_copy(uidx_hbm.at[pl.ds(base, b_per_w)], idx_u)
        pltpu.sync_copy(iidx_hbm.at[pl.ds(base, b_per_w)], idx_i)

        lanes = lax.iota(jnp.int32, 16)
        trq = [(lanes + 16 * q) >> 3 for q in range(hidden // 16)]
        sq = [(lanes + 16 * q) & 7 for q in range(hidden // 16)]

        def fire(uid, iid, d):
            uoff = pl.multiple_of((uid >> 7) * 128, 128)
            ioff = pl.multiple_of((iid >> 7) * 128, 128)
            pltpu.async_copy(utab_hbm.at[:, :, pl.ds(uoff, 128)], bus[d],
                             sems[d])
            pltpu.async_copy(itab_hbm.at[:, :, pl.ds(ioff, 128)], bis[d],
                             sems[d])

        def wait_pair(d):
            pltpu.make_async_copy(utab_hbm.at[:, :, pl.ds(0, 128)], bus[d],
                                  sems[d]).wait()
            pltpu.make_async_copy(itab_hbm.at[:, :, pl.ds(0, 128)], bis[d],
                                  sems[d]).wait()

        def extract(uid, iid, r, d):
            lu = lanes * 0 + (uid & 127)
            li = lanes * 0 + (iid & 127)
            for q in range(hidden // 16):
                vu = plsc.load_gather(bus[d], [trq[q], sq[q], lu])
                vi = plsc.load_gather(bis[d], [trq[q], sq[q], li])
                stage[r, pl.ds(16 * q, 16)] = vu
                stage[r, pl.ds(hidden + 16 * q, 16)] = vi

        def group_body(g, carry):
            vu = idx_u[pl.ds(g * _GRP, _GRP)]
            vi = idx_i[pl.ds(g * _GRP, _GRP)]
            uids = [vu[l] for l in range(_GRP)]
            iids = [vi[l] for l in range(_GRP)]
            for l in range(_DEPTH):
                fire(uids[l], iids[l], l)
            for l in range(_GRP):
                d = l % _DEPTH
                wait_pair(d)
                extract(uids[l], iids[l], (g * _GRP + l) & (_FLUSH - 1), d)
                if l + _DEPTH < _GRP:
                    fire(uids[l + _DEPTH], iids[l + _DEPTH], d)

            @pl.when((g & 3) == 3)
            def _():
                row0 = pl.multiple_of(base + ((g >> 2) << 6), _FLUSH)
                pltpu.sync_copy(stage, out_hbm.at[pl.ds(row0, _FLUSH), :])

            return carry

        lax.fori_loop(0, n_grp, group_body, 0)

    return gather_kernel(uidx, iidx, utab3, itab3)


def _mlp_body(x_ref, w1_ref, b1_ref, w2_ref, b2_ref, w3_ref, b3_ref, o_ref):
    h = jnp.dot(x_ref[...], w1_ref[...], preferred_element_type=jnp.float32)
    h = jnp.maximum(h + b1_ref[...], 0.0)
    h = jnp.maximum(
        jnp.dot(h, w2_ref[...], preferred_element_type=jnp.float32)
        + b2_ref[...], 0.0)
    o_ref[...] = (jnp.dot(h, w3_ref[...], preferred_element_type=jnp.float32)
                  + b3_ref[...])


def _tc_mlp(x, w1, b1r, w2, b2r, w3, b3r, blk):
    batch, d_in = x.shape
    d1 = w1.shape[1]
    d2 = w2.shape[1]
    d3 = w3.shape[1]
    grid = (batch // blk,)
    fixed = lambda b: (0, 0)
    return pl.pallas_call(
        _mlp_body,
        grid=grid,
        in_specs=[
            pl.BlockSpec((blk, d_in), lambda b: (b, 0)),
            pl.BlockSpec((d_in, d1), fixed),
            pl.BlockSpec((1, d1), fixed),
            pl.BlockSpec((d1, d2), fixed),
            pl.BlockSpec((1, d2), fixed),
            pl.BlockSpec((d2, d3), fixed),
            pl.BlockSpec((1, d3), fixed),
        ],
        out_specs=pl.BlockSpec((blk, d3), lambda b: (b, 0)),
        out_shape=jax.ShapeDtypeStruct((batch, d3), jnp.float32),
    )(x, w1, b1r, w2, b2r, w3, b3r)


def kernel(user_id, item_id, user_table, item_table, W1, b1, W2, b2, W3, b3):
    n_ids, hidden = user_table.shape
    utab3 = user_table.T.reshape(8, hidden // 8, n_ids)
    itab3 = item_table.T.reshape(8, hidden // 8, n_ids)
    x = _sc_gather_concat(
        user_id.astype(jnp.int32), item_id.astype(jnp.int32), utab3, itab3)
    return _tc_mlp(
        x, W1, b1.reshape(1, -1), W2, b2.reshape(1, -1),
        W3, b3.reshape(1, -1), blk=2048)


# speedup vs baseline: 2.6183x; 1.0036x over previous
import functools

import jax
import jax.numpy as jnp
from jax import lax
from jax.experimental import pallas as pl
from jax.experimental.pallas import tpu as pltpu
from jax.experimental.pallas import tpu_sc as plsc

_NC = 2
_NS = 16
_NW = _NC * _NS
_GRP = 16
_DEPTH = 4
_FLUSH = 64


def _sc_gather_concat(uidx, iidx, utab3, itab3):
    batch = uidx.shape[0]
    n_tr, n_sub, _ = utab3.shape
    hidden = n_tr * n_sub
    b_per_w = batch // _NW
    n_grp = b_per_w // _GRP
    mesh = plsc.VectorSubcoreMesh(core_axis_name="c", subcore_axis_name="s")

    block = pltpu.VMEM((n_tr, n_sub, 128), jnp.float32)

    @functools.partial(
        pl.kernel,
        mesh=mesh,
        compiler_params=pltpu.CompilerParams(needs_layout_passes=False),
        out_type=jax.ShapeDtypeStruct((batch, 2 * hidden), jnp.float32),
        scratch_types=(
            pltpu.VMEM((b_per_w,), jnp.int32),
            pltpu.VMEM((b_per_w,), jnp.int32),
            [block] * _DEPTH,
            [block] * _DEPTH,
            pltpu.VMEM((_FLUSH, 2 * hidden), jnp.float32),
            [pltpu.SemaphoreType.DMA] * _DEPTH,
        ),
    )
    def gather_kernel(uidx_hbm, iidx_hbm, utab_hbm, itab_hbm, out_hbm,
                      idx_u, idx_i, bus, bis, stage, sems):
        wid = lax.axis_index("s") * _NC + lax.axis_index("c")
        base = wid * b_per_w
        pltpu.sync_copy(uidx_hbm.at[pl.ds(base, b_per_w)], idx_u)
        pltpu.sync_copy(iidx_hbm.at[pl.ds(base, b_per_w)], idx_i)

        lanes = lax.iota(jnp.int32, 16)
        trq = [(lanes + 16 * q) >> 3 for q in range(hidden // 16)]
        sq = [(lanes + 16 * q) & 7 for q in range(hidden // 16)]

        def fire(uid, iid, d):
            uoff = pl.multiple_of((uid >> 7) * 128, 128)
            ioff = pl.multiple_of((iid >> 7) * 128, 128)
            pltpu.async_copy(utab_hbm.at[:, :, pl.ds(uoff, 128)], bus[d],
                             sems[d])
            pltpu.async_copy(itab_hbm.at[:, :, pl.ds(ioff, 128)], bis[d],
                             sems[d])

        def wait_pair(d):
            pltpu.make_async_copy(utab_hbm.at[:, :, pl.ds(0, 128)], bus[d],
                                  sems[d]).wait()
            pltpu.make_async_copy(itab_hbm.at[:, :, pl.ds(0, 128)], bis[d],
                                  sems[d]).wait()

        def extract(uid, iid, r, d):
            lu = lanes * 0 + (uid & 127)
            li = lanes * 0 + (iid & 127)
            for q in range(hidden // 16):
                vu = plsc.load_gather(bus[d], [trq[q], sq[q], lu])
                vi = plsc.load_gather(bis[d], [trq[q], sq[q], li])
                stage[r, pl.ds(16 * q, 16)] = vu
                stage[r, pl.ds(hidden + 16 * q, 16)] = vi

        def load_ids(g):
            vu = idx_u[pl.ds(g * _GRP, _GRP)]
            vi = idx_i[pl.ds(g * _GRP, _GRP)]
            return ([vu[l] for l in range(_GRP)],
                    [vi[l] for l in range(_GRP)])

        uids0, iids0 = load_ids(0)
        for l in range(_DEPTH):
            fire(uids0[l], iids0[l], l)

        def group_body(g, carry):
            uids, iids = load_ids(g)
            gnext = jnp.minimum(g + 1, n_grp - 1)
            uids1, iids1 = load_ids(gnext)
            for l in range(_GRP):
                d = l % _DEPTH
                wait_pair(d)
                extract(uids[l], iids[l], (g * _GRP + l) & (_FLUSH - 1), d)
                if l + _DEPTH < _GRP:
                    fire(uids[l + _DEPTH], iids[l + _DEPTH], d)
                else:
                    ln = l + _DEPTH - _GRP

                    @pl.when(g < n_grp - 1)
                    def _():
                        fire(uids1[ln], iids1[ln], d)

            @pl.when((g & 3) == 3)
            def _():
                row0 = pl.multiple_of(base + ((g >> 2) << 6), _FLUSH)
                pltpu.sync_copy(stage, out_hbm.at[pl.ds(row0, _FLUSH), :])

            return carry

        lax.fori_loop(0, n_grp, group_body, 0)

    return gather_kernel(uidx, iidx, utab3, itab3)


def _mlp_body(x_ref, w1_ref, b1_ref, w2_ref, b2_ref, w3_ref, b3_ref, o_ref):
    h = jnp.dot(x_ref[...], w1_ref[...], preferred_element_type=jnp.float32)
    h = jnp.maximum(h + b1_ref[...], 0.0)
    h = jnp.maximum(
        jnp.dot(h, w2_ref[...], preferred_element_type=jnp.float32)
        + b2_ref[...], 0.0)
    o_ref[...] = (jnp.dot(h, w3_ref[...], preferred_element_type=jnp.float32)
                  + b3_ref[...])


def _tc_mlp(x, w1, b1r, w2, b2r, w3, b3r, blk):
    batch, d_in = x.shape
    d1 = w1.shape[1]
    d2 = w2.shape[1]
    d3 = w3.shape[1]
    grid = (batch // blk,)
    fixed = lambda b: (0, 0)
    return pl.pallas_call(
        _mlp_body,
        grid=grid,
        in_specs=[
            pl.BlockSpec((blk, d_in), lambda b: (b, 0)),
            pl.BlockSpec((d_in, d1), fixed),
            pl.BlockSpec((1, d1), fixed),
            pl.BlockSpec((d1, d2), fixed),
            pl.BlockSpec((1, d2), fixed),
            pl.BlockSpec((d2, d3), fixed),
            pl.BlockSpec((1, d3), fixed),
        ],
        out_specs=pl.BlockSpec((blk, d3), lambda b: (b, 0)),
        out_shape=jax.ShapeDtypeStruct((batch, d3), jnp.float32),
    )(x, w1, b1r, w2, b2r, w3, b3r)


def kernel(user_id, item_id, user_table, item_table, W1, b1, W2, b2, W3, b3):
    n_ids, hidden = user_table.shape
    utab3 = user_table.T.reshape(8, hidden // 8, n_ids)
    itab3 = item_table.T.reshape(8, hidden // 8, n_ids)
    x = _sc_gather_concat(
        user_id.astype(jnp.int32), item_id.astype(jnp.int32), utab3, itab3)
    return _tc_mlp(
        x, W1, b1.reshape(1, -1), W2, b2.reshape(1, -1),
        W3, b3.reshape(1, -1), blk=2048)
